# baseline (speedup 1.0000x reference)
import functools

import jax
import jax.numpy as jnp
from jax import lax
from jax.experimental import pallas as pl
from jax.experimental.pallas import tpu as pltpu
from jax.experimental.pallas import tpu_sc as plsc

DIM = 64
EPS = 1e-6
NC = 2
NS = 16
NW = NC * NS
LANES = 16

CH = 1024
STREAM = 128


def _lane_perm(x, idx):
    return lax.gather(
        x, idx[:, None],
        dimension_numbers=lax.GatherDimensionNumbers(
            offset_dims=(), collapsed_slice_dims=(0,), start_index_map=(0,)),
        slice_sizes=(1,),
        mode=lax.GatherScatterMode.PROMISE_IN_BOUNDS)


def _lane_sum(x, perms):
    for p in perms:
        x = x + _lane_perm(x, p)
    return x


def _rsqrt_vec(m):
    t = jnp.full_like(m, 1.0)
    for _ in range(6):
        t = 0.5 * (t + m / t)
    return 1.0 / t


def kernel(input_ids, table, scale):
    B0, S = input_ids.shape
    V, D = table.shape
    B = B0 * S
    assert D == DIM
    assert B % (NW * CH) == 0
    b_per_w = B // NW
    n_ch = b_per_w // CH
    ids_flat = input_ids.reshape(B)

    mesh = plsc.VectorSubcoreMesh(core_axis_name="c", subcore_axis_name="s")

    @functools.partial(
        pl.kernel,
        mesh=mesh,
        out_type=jax.ShapeDtypeStruct((B, D), jnp.float32),
        scratch_types=[
            pltpu.VMEM((CH,), jnp.int32),
            pltpu.VMEM((CH, D), jnp.float32),
            pltpu.VMEM((D,), jnp.float32),
            pltpu.SemaphoreType.DMA,
        ],
        compiler_params=pltpu.CompilerParams(use_tc_tiling_on_sc=False),
    )
    def sc_kernel(ids_hbm, table_hbm, scale_hbm, out_hbm, idx_v, rows_v,
                  scale_v, sem):
        wid = lax.axis_index("s") * NC + lax.axis_index("c")
        base = wid * b_per_w
        pltpu.sync_copy(scale_hbm, scale_v)
        sc = [scale_v[pl.ds(k * LANES, LANES)] for k in range(D // LANES)]
        lane = lax.iota(jnp.int32, LANES)
        perms = [lane ^ d for d in (1, 2, 4, 8)]

        def chunk_body(ci, _):
            off = base + ci * CH
            pltpu.sync_copy(ids_hbm.at[pl.ds(off, CH)], idx_v)
            copies = [
                pltpu.async_copy(
                    table_hbm.at[idx_v.at[pl.ds(j * STREAM, STREAM)]],
                    rows_v.at[pl.ds(j * STREAM, STREAM)],
                    sem,
                )
                for j in range(CH // STREAM)
            ]
            for c in copies:
                c.wait()

            def row_body(r, _):
                x = [rows_v[r, pl.ds(k * LANES, LANES)]
                     for k in range(D // LANES)]
                s = x[0] * x[0]
                for k in range(1, D // LANES):
                    s = s + x[k] * x[k]
                m = _lane_sum(s, perms) * (1.0 / D) + EPS
                y = _rsqrt_vec(m)
                for k in range(D // LANES):
                    rows_v[r, pl.ds(k * LANES, LANES)] = x[k] * y * sc[k]
                return 0

            lax.fori_loop(0, CH, row_body, 0)
            pltpu.sync_copy(rows_v, out_hbm.at[pl.ds(off, CH)])
            return 0

        lax.fori_loop(0, n_ch, chunk_body, 0)

    out = sc_kernel(ids_flat, table, scale)
    return out.reshape(B0, S, D)


# speedup vs baseline: 2.0207x; 2.0207x over previous
import functools

import jax
import jax.numpy as jnp
from jax import lax
from jax.experimental import pallas as pl
from jax.experimental.pallas import tpu as pltpu
from jax.experimental.pallas import tpu_sc as plsc

DIM = 64
EPS = 1e-6
NC = 2
NS = 16
NW = NC * NS
LANES = 16

CH = 256
NBUF = 4
STREAM = 128
RB = 8


def _lane_perm(x, idx):
    return lax.gather(
        x, idx[:, None],
        dimension_numbers=lax.GatherDimensionNumbers(
            offset_dims=(), collapsed_slice_dims=(0,), start_index_map=(0,)),
        slice_sizes=(1,),
        mode=lax.GatherScatterMode.PROMISE_IN_BOUNDS)


def _rsqrt_vec(m):
    t = 0.59 + 0.417 * m
    for _ in range(4):
        t = 0.5 * (t + m / t)
    return 1.0 / t


def kernel(input_ids, table, scale):
    B0, S = input_ids.shape
    V, D = table.shape
    B = B0 * S
    assert D == DIM
    assert B % (NW * CH * NBUF) == 0
    b_per_w = B // NW
    n_ch = b_per_w // CH
    n_grp = n_ch // NBUF
    ids_flat = input_ids.reshape(B)

    mesh = plsc.VectorSubcoreMesh(core_axis_name="c", subcore_axis_name="s")

    @functools.partial(
        pl.kernel,
        mesh=mesh,
        out_type=jax.ShapeDtypeStruct((B, D), jnp.float32),
        scratch_types=[
            pltpu.VMEM((b_per_w,), jnp.int32),
            pltpu.VMEM((NBUF, CH, D), jnp.float32),
            pltpu.VMEM((RB * 17,), jnp.float32),
            pltpu.VMEM((D,), jnp.float32),
            pltpu.SemaphoreType.DMA((NBUF,)),
            pltpu.SemaphoreType.DMA((NBUF,)),
        ],
        compiler_params=pltpu.CompilerParams(use_tc_tiling_on_sc=False, needs_layout_passes=False),
    )
    def sc_kernel(ids_hbm, table_hbm, scale_hbm, out_hbm, idx_v, rows_v,
                  smat, scale_v, gsem, osem):
        wid = lax.axis_index("s") * NC + lax.axis_index("c")
        base = wid * b_per_w
        pltpu.sync_copy(scale_hbm, scale_v)
        pltpu.sync_copy(ids_hbm.at[pl.ds(base, b_per_w)], idx_v)
        sc = [scale_v[pl.ds(k * LANES, LANES)] for k in range(D // LANES)]
        lane = lax.iota(jnp.int32, LANES)
        fold8 = lane ^ 8
        flatbase = (lane & 7) * 17 + (lane & 8)
        splats = [jnp.full((LANES,), r, jnp.int32) for r in range(RB)]

        def fire_gather(ci, b):
            for j in range(CH // STREAM):
                pltpu.async_copy(
                    table_hbm.at[idx_v.at[pl.ds(ci * CH + j * STREAM, STREAM)]],
                    rows_v.at[b].at[pl.ds(j * STREAM, STREAM)],
                    gsem.at[b],
                )

        def drain_gather(ci, b):
            for j in range(CH // STREAM):
                pltpu.make_async_copy(
                    table_hbm.at[idx_v.at[pl.ds(ci * CH + j * STREAM, STREAM)]],
                    rows_v.at[b].at[pl.ds(j * STREAM, STREAM)],
                    gsem.at[b],
                ).wait()

        def fire_wb(ci, b):
            pltpu.async_copy(rows_v.at[b], out_hbm.at[pl.ds(base + ci * CH, CH)],
                             osem.at[b])

        def drain_wb(ci, b):
            pltpu.make_async_copy(rows_v.at[b],
                                  out_hbm.at[pl.ds(base + ci * CH, CH)],
                                  osem.at[b]).wait()

        def compute(b):
            rows_b = rows_v.at[b]

            def block_body(bi, _):
                r0 = bi * RB
                for r in range(RB):
                    x = [rows_b[r0 + r, pl.ds(k * LANES, LANES)]
                         for k in range(D // LANES)]
                    s = x[0] * x[0]
                    for k in range(1, D // LANES):
                        s = s + x[k] * x[k]
                    smat[pl.ds(r * 17, LANES)] = s
                t = plsc.load_gather(smat, [flatbase])
                for c in range(1, RB):
                    t = t + plsc.load_gather(smat, [flatbase + c])
                m = (t + _lane_perm(t, fold8)) * (1.0 / D) + EPS
                y = _rsqrt_vec(m)
                for r in range(RB):
                    yb = _lane_perm(y, splats[r])
                    for k in range(D // LANES):
                        sl = pl.ds(k * LANES, LANES)
                        rows_b[r0 + r, sl] = rows_b[r0 + r, sl] * yb * sc[k]
                return 0

            lax.fori_loop(0, CH // RB, block_body, 0)

        def group(g, first, last):
            for db in range(NBUF):
                ci = g * NBUF + db
                if first and db < 2:
                    pass
                else:
                    drain_wb(ci - 2, (db - 2) % NBUF)
                if last and db == NBUF - 1:
                    pass
                else:
                    fire_gather(ci + 1, (db + 1) % NBUF)
                drain_gather(ci, db)
                compute(db)
                fire_wb(ci, db)

        fire_gather(0, 0)
        group(0, True, False)

        def grp_body(g, _):
            group(g, False, False)
            return 0

        lax.fori_loop(1, n_grp - 1, grp_body, 0)
        group(n_grp - 1, False, True)
        drain_wb(n_ch - 2, (NBUF - 2) % NBUF)
        drain_wb(n_ch - 1, NBUF - 1)

    out = sc_kernel(ids_flat, table, scale)
    return out.reshape(B0, S, D)


# speedup vs baseline: 2.4119x; 1.1936x over previous
import functools

import jax
import jax.numpy as jnp
from jax import lax
from jax.experimental import pallas as pl
from jax.experimental.pallas import tpu as pltpu
from jax.experimental.pallas import tpu_sc as plsc

DIM = 64
EPS = 1e-6
NC = 2
NS = 16
NW = NC * NS
LANES = 16

CH = 256
NBUF = 4
STREAM = 128
RB = 8


def _lane_perm(x, idx):
    return lax.gather(
        x, idx[:, None],
        dimension_numbers=lax.GatherDimensionNumbers(
            offset_dims=(), collapsed_slice_dims=(0,), start_index_map=(0,)),
        slice_sizes=(1,),
        mode=lax.GatherScatterMode.PROMISE_IN_BOUNDS)


def _rsqrt_vec(m):
    t = 0.59 + 0.417 * m
    for _ in range(4):
        t = 0.5 * (t + m / t)
    return 1.0 / t


def kernel(input_ids, table, scale):
    B0, S = input_ids.shape
    V, D = table.shape
    B = B0 * S
    assert D == DIM
    assert B % (NW * CH * NBUF) == 0
    b_per_w = B // NW
    n_ch = b_per_w // CH
    n_grp = n_ch // NBUF
    ids_flat = input_ids.reshape(B)

    mesh = plsc.VectorSubcoreMesh(core_axis_name="c", subcore_axis_name="s")

    @functools.partial(
        pl.kernel,
        mesh=mesh,
        out_type=jax.ShapeDtypeStruct((B, D), jnp.float32),
        scratch_types=[
            pltpu.VMEM((b_per_w,), jnp.int32),
            pltpu.VMEM((NBUF, CH, D), jnp.float32),
            pltpu.VMEM((CH // RB, RB * 17), jnp.float32),
            pltpu.VMEM((D,), jnp.float32),
            pltpu.SemaphoreType.DMA((NBUF,)),
            pltpu.SemaphoreType.DMA((NBUF,)),
        ],
        compiler_params=pltpu.CompilerParams(use_tc_tiling_on_sc=False, needs_layout_passes=False),
    )
    def sc_kernel(ids_hbm, table_hbm, scale_hbm, out_hbm, idx_v, rows_v,
                  smat, scale_v, gsem, osem):
        wid = lax.axis_index("s") * NC + lax.axis_index("c")
        base = wid * b_per_w
        pltpu.sync_copy(scale_hbm, scale_v)
        pltpu.sync_copy(ids_hbm.at[pl.ds(base, b_per_w)], idx_v)
        sc = [scale_v[pl.ds(k * LANES, LANES)] for k in range(D // LANES)]
        lane = lax.iota(jnp.int32, LANES)
        fold8 = lane ^ 8
        flatbase = (lane & 7) * 17 + (lane & 8)
        splats = [jnp.full((LANES,), r, jnp.int32) for r in range(RB)]

        def fire_gather(ci, b):
            for j in range(CH // STREAM):
                pltpu.async_copy(
                    table_hbm.at[idx_v.at[pl.ds(ci * CH + j * STREAM, STREAM)]],
                    rows_v.at[b].at[pl.ds(j * STREAM, STREAM)],
                    gsem.at[b],
                )

        def drain_gather(ci, b):
            for j in range(CH // STREAM):
                pltpu.make_async_copy(
                    table_hbm.at[idx_v.at[pl.ds(ci * CH + j * STREAM, STREAM)]],
                    rows_v.at[b].at[pl.ds(j * STREAM, STREAM)],
                    gsem.at[b],
                ).wait()

        def fire_wb(ci, b):
            pltpu.async_copy(rows_v.at[b], out_hbm.at[pl.ds(base + ci * CH, CH)],
                             osem.at[b])

        def drain_wb(ci, b):
            pltpu.make_async_copy(rows_v.at[b],
                                  out_hbm.at[pl.ds(base + ci * CH, CH)],
                                  osem.at[b]).wait()

        def compute(b):
            rows_b = rows_v.at[b]

            @plsc.parallel_loop(0, CH // RB, unroll=4)
            def block_body(bi):
                r0 = bi * RB
                xs = []
                for r in range(RB):
                    x = [rows_b[r0 + r, pl.ds(k * LANES, LANES)]
                         for k in range(D // LANES)]
                    s = x[0] * x[0]
                    for k in range(1, D // LANES):
                        s = s + x[k] * x[k]
                    smat[bi, pl.ds(r * 17, LANES)] = s
                t = plsc.load_gather(smat.at[bi], [flatbase])
                for c in range(1, RB):
                    t = t + plsc.load_gather(smat.at[bi], [flatbase + c])
                m = (t + _lane_perm(t, fold8)) * (1.0 / D) + EPS
                y = _rsqrt_vec(m)
                for r in range(RB):
                    yb = _lane_perm(y, splats[r])
                    for k in range(D // LANES):
                        sl = pl.ds(k * LANES, LANES)
                        rows_b[r0 + r, sl] = rows_b[r0 + r, sl] * yb * sc[k]

        fire_gather(0, 0)

        def grp_body(g, _):
            for db in range(NBUF):
                ci = g * NBUF + db

                @pl.when(ci >= 2)
                def _():
                    drain_wb(ci - 2, (db - 2) % NBUF)

                @pl.when(ci + 1 < n_ch)
                def _():
                    fire_gather(ci + 1, (db + 1) % NBUF)

                drain_gather(ci, db)
                compute(db)
                fire_wb(ci, db)
            return 0

        lax.fori_loop(0, n_grp, grp_body, 0)
        drain_wb(n_ch - 2, NBUF - 2)
        drain_wb(n_ch - 1, NBUF - 1)

    out = sc_kernel(ids_flat, table, scale)
    return out.reshape(B0, S, D)


# speedup vs baseline: 2.4766x; 1.0268x over previous
import functools

import jax
import jax.numpy as jnp
from jax import lax
from jax.experimental import pallas as pl
from jax.experimental.pallas import tpu as pltpu
from jax.experimental.pallas import tpu_sc as plsc

DIM = 64
EPS = 1e-6
NC = 2
NS = 16
NW = NC * NS
LANES = 16

CH = 1024
STREAM = 128


def _lane_perm(x, idx):
    return lax.gather(
        x, idx[:, None],
        dimension_numbers=lax.GatherDimensionNumbers(
            offset_dims=(), collapsed_slice_dims=(0,), start_index_map=(0,)),
        slice_sizes=(1,),
        mode=lax.GatherScatterMode.PROMISE_IN_BOUNDS)


def _lane_sum(x, perms):
    for p in perms:
        x = x + _lane_perm(x, p)
    return x


def _rsqrt_vec(m):
    t = jnp.full_like(m, 1.0)
    for _ in range(6):
        t = 0.5 * (t + m / t)
    return 1.0 / t


def kernel(input_ids, table, scale):
    B0, S = input_ids.shape
    V, D = table.shape
    B = B0 * S
    assert D == DIM
    assert B % (NW * CH) == 0
    b_per_w = B // NW
    n_ch = b_per_w // CH
    ids_flat = input_ids.reshape(B)

    mesh = plsc.VectorSubcoreMesh(core_axis_name="c", subcore_axis_name="s")

    @functools.partial(
        pl.kernel,
        mesh=mesh,
        out_type=jax.ShapeDtypeStruct((B, D), jnp.float32),
        scratch_types=[
            pltpu.VMEM((CH,), jnp.int32),
            pltpu.VMEM((CH, D), jnp.float32),
            pltpu.VMEM((D,), jnp.float32),
            pltpu.SemaphoreType.DMA,
        ],
        compiler_params=pltpu.CompilerParams(use_tc_tiling_on_sc=False),
    )
    def sc_kernel(ids_hbm, table_hbm, scale_hbm, out_hbm, idx_v, rows_v,
                  scale_v, sem):
        wid = lax.axis_index("s") * NC + lax.axis_index("c")
        base = wid * b_per_w
        pltpu.sync_copy(scale_hbm, scale_v)
        sc = [scale_v[pl.ds(k * LANES, LANES)] for k in range(D // LANES)]
        lane = lax.iota(jnp.int32, LANES)
        perms = [lane ^ d for d in (1, 2, 4, 8)]

        def chunk_body(ci, _):
            off = base + ci * CH
            pltpu.sync_copy(ids_hbm.at[pl.ds(off, CH)], idx_v)
            copies = [
                pltpu.async_copy(
                    table_hbm.at[idx_v.at[pl.ds(j * STREAM, STREAM)]],
                    rows_v.at[pl.ds(j * STREAM, STREAM)],
                    sem,
                )
                for j in range(CH // STREAM)
            ]
            for c in copies:
                c.wait()

            def row_body(r, _):
                x = [rows_v[r, pl.ds(k * LANES, LANES)]
                     for k in range(D // LANES)]
                s = x[0] * x[0]
                for k in range(1, D // LANES):
                    s = s + x[k] * x[k]
                m = _lane_sum(s, perms) * (1.0 / D) + EPS
                y = _rsqrt_vec(m)
                for k in range(D // LANES):
                    rows_v[r, pl.ds(k * LANES, LANES)] = x[k] * y * sc[k]
                return 0

            lax.fori_loop(0, CH, row_body, 0)
            pltpu.sync_copy(rows_v, out_hbm.at[pl.ds(off, CH)])
            return 0

        lax.fori_loop(0, n_ch, chunk_body, 0)

    out = sc_kernel(ids_flat, table, scale)
    return out.reshape(B0, S, D)
